# baseline (speedup 1.0000x reference)
import jax
import jax.numpy as jnp
from jax import lax
from jax.experimental import pallas as pl
from jax.experimental.pallas import tpu as pltpu
from jax.experimental.pallas import tpu_sc as plsc

NN = 2560
NE = 5120
NG = 128
NPG = NN // NG
TP = 8
H = 64
ACT = 6
NC = 2
NS = 16
NW = NC * NS
W128 = 128
EBLK = 512


def _relu(v):
    return jnp.maximum(v, 0.0)


def _dot(a, b):
    return jnp.dot(a, b, preferred_element_type=jnp.float32,
                   precision=lax.Precision.HIGHEST)


def _sc_gather(table, idx, width):
    B = idx.shape[0]
    bpw = B // NW
    mesh = plsc.VectorSubcoreMesh(core_axis_name="c", subcore_axis_name="s")

    def body(table_hbm, idx_hbm, out_hbm, idx_v, rows_v, sem):
        wid = lax.axis_index("s") * NC + lax.axis_index("c")
        base = wid * bpw
        pltpu.sync_copy(idx_hbm.at[pl.ds(base, bpw)], idx_v)
        pltpu.async_copy(table_hbm.at[idx_v], rows_v, sem).wait()
        pltpu.sync_copy(rows_v, out_hbm.at[pl.ds(base, bpw)])

    f = pl.kernel(
        body,
        out_type=jax.ShapeDtypeStruct((B, width), jnp.float32),
        mesh=mesh,
        scratch_types=[
            pltpu.VMEM((bpw,), jnp.int32),
            pltpu.VMEM((bpw, width), jnp.float32),
            pltpu.SemaphoreType.DMA,
        ],
    )
    return f(table, idx)


def _sc_scatter_add(values, idx, zeros_rows):
    B, width = values.shape
    n_rows = zeros_rows.shape[0]
    bpw = B // NW
    rpw = n_rows // NS
    mesh = plsc.VectorSubcoreMesh(core_axis_name="c", subcore_axis_name="s")

    def body(val_hbm, idx_hbm, zero_hbm, out_hbm, idx_v, rows_v, acc, sem):
        cid = lax.axis_index("c")
        sid = lax.axis_index("s")
        pltpu.sync_copy(zero_hbm.at[pl.ds(sid * rpw, rpw)],
                        acc.at[pl.ds(sid * rpw, rpw)])
        plsc.subcore_barrier()
        base = (sid * NC + cid) * bpw
        pltpu.sync_copy(idx_hbm.at[pl.ds(base, bpw)], idx_v)
        pltpu.sync_copy(val_hbm.at[pl.ds(base, bpw)], rows_v)
        pltpu.sync_copy(rows_v, acc.at[idx_v], add=True)
        plsc.subcore_barrier()
        pltpu.sync_copy(acc.at[pl.ds(sid * rpw, rpw)],
                        out_hbm.at[cid, pl.ds(sid * rpw, rpw)])

    f = pl.kernel(
        body,
        out_type=jax.ShapeDtypeStruct((NC, n_rows, width), jnp.float32),
        mesh=mesh,
        scratch_types=[
            pltpu.VMEM((bpw,), jnp.int32),
            pltpu.VMEM((bpw, width), jnp.float32),
            pltpu.VMEM_SHARED((n_rows, width), jnp.float32),
            pltpu.SemaphoreType.DMA,
        ],
    )
    return f(values, idx, zeros_rows)


def _tc_prep(x, W_lin0, b_lin0, edge_attr, We1, be1):
    def body(x_ref, wl, bl, ea, we1, be1_, out_ref, f_ref):
        s = _relu(_dot(x_ref[...], wl[...]) + bl[...])
        out_ref[...] = jnp.concatenate([s, jnp.zeros((NN, W128 - H), jnp.float32)],
                                       axis=1)
        f_ref[...] = _relu(_dot(ea[...], we1[...]) + be1_[...])

    return pl.pallas_call(
        body,
        out_shape=(jax.ShapeDtypeStruct((NN, W128), jnp.float32),
                   jax.ShapeDtypeStruct((NE, H), jnp.float32)),
    )(x, W_lin0, b_lin0, edge_attr, We1, be1)


def _tc_msg(f, o_src, M, Be2):
    def body(f_ref, o_ref, m_ref, b_ref, msg_ref):
        fb = f_ref[...]
        ob = o_ref[...][:, :H]
        z = (fb[:, :, None] * ob[:, None, :]).reshape(EBLK, H * H)
        msg = _dot(z, m_ref[...]) + _dot(ob, b_ref[...])
        col = lax.broadcasted_iota(jnp.int32, (EBLK, W128 - H), 1)
        pad = jnp.where(col == 0, 1.0, 0.0)
        msg_ref[...] = jnp.concatenate([msg, pad], axis=1)

    return pl.pallas_call(
        body,
        grid=(NE // EBLK,),
        in_specs=[
            pl.BlockSpec((EBLK, H), lambda i: (i, 0)),
            pl.BlockSpec((EBLK, W128), lambda i: (i, 0)),
            pl.BlockSpec((H * H, H), lambda i: (0, 0)),
            pl.BlockSpec((H, H), lambda i: (0, 0)),
        ],
        out_specs=pl.BlockSpec((EBLK, W128), lambda i: (i, 0)),
        out_shape=jax.ShapeDtypeStruct((NE, W128), jnp.float32),
    )(f, o_src, M, Be2)


def _tc_gru(s, agg2, W_root, b_conv, Wih, bih, Whh, bhh):
    R = s.shape[0]

    def body(s_ref, agg_ref, wr, bc, wih, bih_, whh, bhh_, out_ref):
        ag = agg_ref[...]
        both = ag[0] + ag[1]
        agg = both[:, :H]
        deg = both[:, H:H + 1]
        inv = 1.0 / jnp.maximum(deg, 1.0)
        s_ = s_ref[...][:, :H]
        m = _relu(_dot(s_, wr[...]) + agg * inv + bc[...])
        gx = _dot(m, wih[...]) + bih_[...]
        gh = _dot(s_, whh[...]) + bhh_[...]
        r = jax.nn.sigmoid(gx[:, :H] + gh[:, :H])
        zg = jax.nn.sigmoid(gx[:, H:2 * H] + gh[:, H:2 * H])
        n = jnp.tanh(gx[:, 2 * H:] + r * gh[:, 2 * H:])
        s_new = (1.0 - zg) * n + zg * s_
        out_ref[...] = jnp.concatenate(
            [s_new, jnp.zeros((R, W128 - H), jnp.float32)], axis=1)

    return pl.pallas_call(
        body,
        out_shape=jax.ShapeDtypeStruct((R, W128), jnp.float32),
    )(s, agg2, W_root, b_conv, Wih, bih, Whh, bhh)


def _tc_final(out_nodes, n_feat, Wih_s2s, Whh_s2s, b_s2s, Wih_m, b_m,
              Wm1a, Wm1b, bm1, Wm2, bm2):
    def body(o_ref, nf_ref, wihs, whhs, bs, wim, bm, w1a, w1b, b1, w2, b2,
             logit_ref, hx_ref, cx_ref):
        o3 = o_ref[...][:, :H].reshape(NG, NPG, H)
        q_star = jnp.zeros((NG, 2 * H), jnp.float32)
        hs = jnp.zeros((NG, H), jnp.float32)
        cs = jnp.zeros((NG, H), jnp.float32)
        for _ in range(6):
            gates = _dot(q_star, wihs[...]) + _dot(hs, whhs[...]) + bs[...]
            ig = jax.nn.sigmoid(gates[:, :H])
            fg = jax.nn.sigmoid(gates[:, H:2 * H])
            gg = jnp.tanh(gates[:, 2 * H:3 * H])
            og = jax.nn.sigmoid(gates[:, 3 * H:])
            cs = fg * cs + ig * gg
            hs = og * jnp.tanh(cs)
            e = jnp.sum(o3 * hs[:, None, :], axis=2)
            ex = jnp.exp(e - jnp.max(e, axis=1, keepdims=True))
            a = ex / jnp.sum(ex, axis=1, keepdims=True)
            rg = jnp.sum(a[:, :, None] * o3, axis=1)
            q_star = jnp.concatenate([hs, rg], axis=1)
        gates = _dot(q_star, wim[...]) + bm[...]
        ig = jax.nn.sigmoid(gates[:, :H])
        gg = jnp.tanh(gates[:, 2 * H:3 * H])
        og = jax.nn.sigmoid(gates[:, 3 * H:])
        cx = ig * gg
        hx = og * jnp.tanh(cx)
        hx_ref[...] = hx
        cx_ref[...] = cx
        t = _dot(hx, w1a[...])
        t4 = jnp.broadcast_to(t[:, None, :], (NG, TP, H)).reshape(NG * TP, H)
        hid = _relu(t4 + _dot(nf_ref[...], w1b[...]) + b1[...])
        logit_ref[...] = _dot(hid, w2[...]) + b2[...]

    return pl.pallas_call(
        body,
        out_shape=(jax.ShapeDtypeStruct((NG * TP, ACT), jnp.float32),
                   jax.ShapeDtypeStruct((NG, H), jnp.float32),
                   jax.ShapeDtypeStruct((NG, H), jnp.float32)),
    )(out_nodes, n_feat, Wih_s2s, Whh_s2s, b_s2s, Wih_m, b_m,
      Wm1a, Wm1b, bm1, Wm2, bm2)


def kernel(x, edge_attr, W_lin0, b_lin0, We1, be1, We2, be2, W_root, b_conv,
           Wih_gru, Whh_gru, bih_gru, bhh_gru, Wih_s2s, Whh_s2s, bih_s2s,
           bhh_s2s, Wih_m, Whh_m, bih_m, bhh_m, Wm1, bm1, Wm2, bm2,
           edge_index, batch, nonring, nrbidx):
    f32 = jnp.float32
    src = edge_index[0]
    dst = edge_index[1]
    M = We2.reshape(H * H, H)
    Be2 = be2.reshape(H, H)
    zeros_agg = jnp.zeros((NN, W128), f32)

    gru_w = (W_root, b_conv.reshape(1, H), Wih_gru,
             bih_gru.reshape(1, 3 * H), Whh_gru, bhh_gru.reshape(1, 3 * H))
    s, f = _tc_prep(x, W_lin0, b_lin0.reshape(1, H), edge_attr, We1,
                    be1.reshape(1, H))
    for t in range(6):
        T = _sc_gather(s, src, W128)
        msg = _tc_msg(f, T, M, Be2)
        agg2 = _sc_scatter_add(msg, dst, zeros_agg)
        s = _tc_gru(s, agg2, *gru_w)
    nf = _sc_gather(s, nonring.reshape(-1), W128)[:, :H].reshape(NG * TP, 4 * H)
    logits, hx, cx = _tc_final(
        s, nf, Wih_s2s, Whh_s2s, (bih_s2s + bhh_s2s).reshape(1, 4 * H),
        Wih_m, (bih_m + bhh_m).reshape(1, 4 * H),
        Wm1[:H], Wm1[H:], bm1.reshape(1, H), Wm2, bm2.reshape(1, ACT))
    return (logits.reshape(NG, TP, ACT), hx[None], cx[None])


# speedup vs baseline: 1.0963x; 1.0963x over previous
import jax
import jax.numpy as jnp
from jax import lax
from jax.experimental import pallas as pl
from jax.experimental.pallas import tpu as pltpu
from jax.experimental.pallas import tpu_sc as plsc

NN = 2560
NE = 5120
NG = 128
NPG = NN // NG
TP = 8
H = 64
ACT = 6
NC = 2
NS = 16
NW = NC * NS
W128 = 128
EBLK = 1024


def _relu(v):
    return jnp.maximum(v, 0.0)


def _dot(a, b):
    return jnp.dot(a, b, preferred_element_type=jnp.float32,
                   precision=lax.Precision.HIGHEST)


def _sc_gather(table, idx, width):
    B = idx.shape[0]
    bpw = B // NW
    mesh = plsc.VectorSubcoreMesh(core_axis_name="c", subcore_axis_name="s")

    def body(table_hbm, idx_hbm, out_hbm, idx_v, rows_v, sem):
        wid = lax.axis_index("s") * NC + lax.axis_index("c")
        base = wid * bpw
        pltpu.sync_copy(idx_hbm.at[pl.ds(base, bpw)], idx_v)
        pltpu.async_copy(table_hbm.at[idx_v], rows_v, sem).wait()
        pltpu.sync_copy(rows_v, out_hbm.at[pl.ds(base, bpw)])

    f = pl.kernel(
        body,
        out_type=jax.ShapeDtypeStruct((B, width), jnp.float32),
        mesh=mesh,
        scratch_types=[
            pltpu.VMEM((bpw,), jnp.int32),
            pltpu.VMEM((bpw, width), jnp.float32),
            pltpu.SemaphoreType.DMA,
        ],
    )
    return f(table, idx)


def _sc_scatter_add(values, idx, zeros_rows):
    B, width = values.shape
    n_rows = zeros_rows.shape[0]
    bpw = B // NW
    rpw = n_rows // NS
    mesh = plsc.VectorSubcoreMesh(core_axis_name="c", subcore_axis_name="s")

    def body(val_hbm, idx_hbm, zero_hbm, out_hbm, idx_v, rows_v, acc, sem):
        cid = lax.axis_index("c")
        sid = lax.axis_index("s")
        pltpu.sync_copy(zero_hbm.at[pl.ds(sid * rpw, rpw)],
                        acc.at[pl.ds(sid * rpw, rpw)])
        plsc.subcore_barrier()
        base = (sid * NC + cid) * bpw
        pltpu.sync_copy(idx_hbm.at[pl.ds(base, bpw)], idx_v)
        pltpu.sync_copy(val_hbm.at[pl.ds(base, bpw)], rows_v)
        pltpu.sync_copy(rows_v, acc.at[idx_v], add=True)
        plsc.subcore_barrier()
        pltpu.sync_copy(acc.at[pl.ds(sid * rpw, rpw)],
                        out_hbm.at[cid, pl.ds(sid * rpw, rpw)])

    f = pl.kernel(
        body,
        out_type=jax.ShapeDtypeStruct((NC, n_rows, width), jnp.float32),
        mesh=mesh,
        scratch_types=[
            pltpu.VMEM((bpw,), jnp.int32),
            pltpu.VMEM((bpw, width), jnp.float32),
            pltpu.VMEM_SHARED((n_rows, width), jnp.float32),
            pltpu.SemaphoreType.DMA,
        ],
    )
    return f(values, idx, zeros_rows)


def _tc_prep(x, W_lin0, b_lin0, edge_attr, We1, be1):
    def body(x_ref, wl, bl, ea, we1, be1_, out_ref, f_ref):
        s = _relu(_dot(x_ref[...], wl[...]) + bl[...])
        out_ref[...] = jnp.concatenate([s, jnp.zeros((NN, W128 - H), jnp.float32)],
                                       axis=1)
        f_ref[...] = _relu(_dot(ea[...], we1[...]) + be1_[...])

    return pl.pallas_call(
        body,
        out_shape=(jax.ShapeDtypeStruct((NN, W128), jnp.float32),
                   jax.ShapeDtypeStruct((NE, H), jnp.float32)),
    )(x, W_lin0, b_lin0, edge_attr, We1, be1)


def _tc_msg(f, o_src, M, Be2):
    def body(f_ref, o_ref, m_ref, b_ref, msg_ref):
        fb = f_ref[...]
        ob = o_ref[...][:, :H]
        z = (fb[:, :, None] * ob[:, None, :]).reshape(EBLK, H * H)
        m_full = m_ref[...]
        z_hi = z.astype(jnp.bfloat16)
        z_lo = (z - z_hi.astype(jnp.float32)).astype(jnp.bfloat16)
        m_hi = m_full.astype(jnp.bfloat16)
        m_lo = (m_full - m_hi.astype(jnp.float32)).astype(jnp.bfloat16)

        def bdot(a, b):
            return jnp.dot(a, b, preferred_element_type=jnp.float32,
                           precision=lax.Precision.DEFAULT)

        msg = (bdot(z_hi, m_lo) + bdot(z_lo, m_hi)) + bdot(z_hi, m_hi)
        msg = msg + _dot(ob, b_ref[...])
        col = lax.broadcasted_iota(jnp.int32, (EBLK, W128 - H), 1)
        pad = jnp.where(col == 0, 1.0, 0.0)
        msg_ref[...] = jnp.concatenate([msg, pad], axis=1)

    return pl.pallas_call(
        body,
        grid=(NE // EBLK,),
        in_specs=[
            pl.BlockSpec((EBLK, H), lambda i: (i, 0)),
            pl.BlockSpec((EBLK, W128), lambda i: (i, 0)),
            pl.BlockSpec((H * H, H), lambda i: (0, 0)),
            pl.BlockSpec((H, H), lambda i: (0, 0)),
        ],
        out_specs=pl.BlockSpec((EBLK, W128), lambda i: (i, 0)),
        out_shape=jax.ShapeDtypeStruct((NE, W128), jnp.float32),
    )(f, o_src, M, Be2)


def _tc_gru(s, agg2, W_root, b_conv, Wih, bih, Whh, bhh):
    R = s.shape[0]

    def body(s_ref, agg_ref, wr, bc, wih, bih_, whh, bhh_, out_ref):
        ag = agg_ref[...]
        both = ag[0] + ag[1]
        agg = both[:, :H]
        deg = both[:, H:H + 1]
        inv = 1.0 / jnp.maximum(deg, 1.0)
        s_ = s_ref[...][:, :H]
        m = _relu(_dot(s_, wr[...]) + agg * inv + bc[...])
        gx = _dot(m, wih[...]) + bih_[...]
        gh = _dot(s_, whh[...]) + bhh_[...]
        r = jax.nn.sigmoid(gx[:, :H] + gh[:, :H])
        zg = jax.nn.sigmoid(gx[:, H:2 * H] + gh[:, H:2 * H])
        n = jnp.tanh(gx[:, 2 * H:] + r * gh[:, 2 * H:])
        s_new = (1.0 - zg) * n + zg * s_
        out_ref[...] = jnp.concatenate(
            [s_new, jnp.zeros((R, W128 - H), jnp.float32)], axis=1)

    return pl.pallas_call(
        body,
        out_shape=jax.ShapeDtypeStruct((R, W128), jnp.float32),
    )(s, agg2, W_root, b_conv, Wih, bih, Whh, bhh)


def _tc_final(out_nodes, n_feat, Wih_s2s, Whh_s2s, b_s2s, Wih_m, b_m,
              Wm1a, Wm1b, bm1, Wm2, bm2):
    def body(o_ref, nf_ref, wihs, whhs, bs, wim, bm, w1a, w1b, b1, w2, b2,
             logit_ref, hx_ref, cx_ref):
        o3 = o_ref[...][:, :H].reshape(NG, NPG, H)
        q_star = jnp.zeros((NG, 2 * H), jnp.float32)
        hs = jnp.zeros((NG, H), jnp.float32)
        cs = jnp.zeros((NG, H), jnp.float32)
        for _ in range(6):
            gates = _dot(q_star, wihs[...]) + _dot(hs, whhs[...]) + bs[...]
            ig = jax.nn.sigmoid(gates[:, :H])
            fg = jax.nn.sigmoid(gates[:, H:2 * H])
            gg = jnp.tanh(gates[:, 2 * H:3 * H])
            og = jax.nn.sigmoid(gates[:, 3 * H:])
            cs = fg * cs + ig * gg
            hs = og * jnp.tanh(cs)
            e = jnp.sum(o3 * hs[:, None, :], axis=2)
            ex = jnp.exp(e - jnp.max(e, axis=1, keepdims=True))
            a = ex / jnp.sum(ex, axis=1, keepdims=True)
            rg = jnp.sum(a[:, :, None] * o3, axis=1)
            q_star = jnp.concatenate([hs, rg], axis=1)
        gates = _dot(q_star, wim[...]) + bm[...]
        ig = jax.nn.sigmoid(gates[:, :H])
        gg = jnp.tanh(gates[:, 2 * H:3 * H])
        og = jax.nn.sigmoid(gates[:, 3 * H:])
        cx = ig * gg
        hx = og * jnp.tanh(cx)
        hx_ref[...] = hx
        cx_ref[...] = cx
        t = _dot(hx, w1a[...])
        t4 = jnp.broadcast_to(t[:, None, :], (NG, TP, H)).reshape(NG * TP, H)
        hid = _relu(t4 + _dot(nf_ref[...], w1b[...]) + b1[...])
        logit_ref[...] = _dot(hid, w2[...]) + b2[...]

    return pl.pallas_call(
        body,
        out_shape=(jax.ShapeDtypeStruct((NG * TP, ACT), jnp.float32),
                   jax.ShapeDtypeStruct((NG, H), jnp.float32),
                   jax.ShapeDtypeStruct((NG, H), jnp.float32)),
    )(out_nodes, n_feat, Wih_s2s, Whh_s2s, b_s2s, Wih_m, b_m,
      Wm1a, Wm1b, bm1, Wm2, bm2)


def kernel(x, edge_attr, W_lin0, b_lin0, We1, be1, We2, be2, W_root, b_conv,
           Wih_gru, Whh_gru, bih_gru, bhh_gru, Wih_s2s, Whh_s2s, bih_s2s,
           bhh_s2s, Wih_m, Whh_m, bih_m, bhh_m, Wm1, bm1, Wm2, bm2,
           edge_index, batch, nonring, nrbidx):
    f32 = jnp.float32
    src = edge_index[0]
    dst = edge_index[1]
    M = We2.reshape(H * H, H)
    Be2 = be2.reshape(H, H)
    zeros_agg = jnp.zeros((NN, W128), f32)

    gru_w = (W_root, b_conv.reshape(1, H), Wih_gru,
             bih_gru.reshape(1, 3 * H), Whh_gru, bhh_gru.reshape(1, 3 * H))
    s, f = _tc_prep(x, W_lin0, b_lin0.reshape(1, H), edge_attr, We1,
                    be1.reshape(1, H))
    for t in range(6):
        T = _sc_gather(s, src, W128)
        msg = _tc_msg(f, T, M, Be2)
        agg2 = _sc_scatter_add(msg, dst, zeros_agg)
        s = _tc_gru(s, agg2, *gru_w)
    nf = _sc_gather(s, nonring.reshape(-1), W128)[:, :H].reshape(NG * TP, 4 * H)
    logits, hx, cx = _tc_final(
        s, nf, Wih_s2s, Whh_s2s, (bih_s2s + bhh_s2s).reshape(1, 4 * H),
        Wih_m, (bih_m + bhh_m).reshape(1, 4 * H),
        Wm1[:H], Wm1[H:], bm1.reshape(1, H), Wm2, bm2.reshape(1, ACT))
    return (logits.reshape(NG, TP, ACT), hx[None], cx[None])


# speedup vs baseline: 1.1083x; 1.0110x over previous
import jax
import jax.numpy as jnp
from jax import lax
from jax.experimental import pallas as pl
from jax.experimental.pallas import tpu as pltpu
from jax.experimental.pallas import tpu_sc as plsc

NN = 2560
NE = 5120
NG = 128
NPG = NN // NG
TP = 8
H = 64
ACT = 6
NC = 2
NS = 16
NW = NC * NS
W128 = 128
EBLK = 1024


def _relu(v):
    return jnp.maximum(v, 0.0)


def _dot(a, b):
    return jnp.dot(a, b, preferred_element_type=jnp.float32,
                   precision=lax.Precision.HIGHEST)


def _sc_gather(table, idx, width):
    B = idx.shape[0]
    bpw = B // NW
    mesh = plsc.VectorSubcoreMesh(core_axis_name="c", subcore_axis_name="s")

    def body(table_hbm, idx_hbm, out_hbm, idx_v, rows_v, sem):
        wid = lax.axis_index("s") * NC + lax.axis_index("c")
        base = wid * bpw
        pltpu.sync_copy(idx_hbm.at[pl.ds(base, bpw)], idx_v)
        pltpu.async_copy(table_hbm.at[idx_v], rows_v, sem).wait()
        pltpu.sync_copy(rows_v, out_hbm.at[pl.ds(base, bpw)])

    f = pl.kernel(
        body,
        out_type=jax.ShapeDtypeStruct((B, width), jnp.float32),
        mesh=mesh,
        scratch_types=[
            pltpu.VMEM((bpw,), jnp.int32),
            pltpu.VMEM((bpw, width), jnp.float32),
            pltpu.SemaphoreType.DMA,
        ],
    )
    return f(table, idx)


def _sc_scatter_add(values, idx, zeros_rows):
    B, width = values.shape
    n_rows = zeros_rows.shape[0]
    bpw = B // NW
    rpw = n_rows // NS
    mesh = plsc.VectorSubcoreMesh(core_axis_name="c", subcore_axis_name="s")

    def body(val_hbm, idx_hbm, zero_hbm, out_hbm, idx_v, rows_v, acc, sem):
        cid = lax.axis_index("c")
        sid = lax.axis_index("s")
        pltpu.sync_copy(zero_hbm.at[pl.ds(sid * rpw, rpw)],
                        acc.at[pl.ds(sid * rpw, rpw)])
        plsc.subcore_barrier()
        base = (sid * NC + cid) * bpw
        pltpu.sync_copy(idx_hbm.at[pl.ds(base, bpw)], idx_v)
        pltpu.sync_copy(val_hbm.at[pl.ds(base, bpw)], rows_v)
        pltpu.sync_copy(rows_v, acc.at[idx_v], add=True)
        plsc.subcore_barrier()
        pltpu.sync_copy(acc.at[pl.ds(sid * rpw, rpw)],
                        out_hbm.at[cid, pl.ds(sid * rpw, rpw)])

    f = pl.kernel(
        body,
        out_type=jax.ShapeDtypeStruct((NC, n_rows, width), jnp.float32),
        mesh=mesh,
        scratch_types=[
            pltpu.VMEM((bpw,), jnp.int32),
            pltpu.VMEM((bpw, width), jnp.float32),
            pltpu.VMEM_SHARED((n_rows, width), jnp.float32),
            pltpu.SemaphoreType.DMA,
        ],
    )
    return f(values, idx, zeros_rows)


def _tc_prep(x, W_lin0, b_lin0, edge_attr, We1, be1):
    def body(x_ref, wl, bl, ea, we1, be1_, out_ref, f_ref):
        s = _relu(_dot(x_ref[...], wl[...]) + bl[...])
        out_ref[...] = jnp.concatenate([s, jnp.zeros((NN, W128 - H), jnp.float32)],
                                       axis=1)
        f_ref[...] = _relu(_dot(ea[...], we1[...]) + be1_[...])

    return pl.pallas_call(
        body,
        out_shape=(jax.ShapeDtypeStruct((NN, W128), jnp.float32),
                   jax.ShapeDtypeStruct((NE, H), jnp.float32)),
    )(x, W_lin0, b_lin0, edge_attr, We1, be1)


def _tc_msg(f, o_src, M, Be2):
    def body(f_ref, o_ref, m_ref, b_ref, msg_ref):
        fb = f_ref[...]
        ob = o_ref[...][:, :H]
        m_full = m_ref[...]
        o_hi = ob.astype(jnp.bfloat16)
        o_lo = (ob - o_hi.astype(jnp.float32)).astype(jnp.bfloat16)
        m_hi = m_full.astype(jnp.bfloat16)
        m_lo = (m_full - m_hi.astype(jnp.float32)).astype(jnp.bfloat16)

        def bdot(a, b):
            return jnp.dot(a, b, preferred_element_type=jnp.float32,
                           precision=lax.Precision.DEFAULT)

        P = (bdot(o_hi, m_lo) + bdot(o_lo, m_hi)) + bdot(o_hi, m_hi)
        msg = jnp.sum(P.reshape(EBLK, H, H) * fb[:, :, None], axis=1)
        msg = msg + _dot(ob, b_ref[...])
        col = lax.broadcasted_iota(jnp.int32, (EBLK, W128 - H), 1)
        pad = jnp.where(col == 0, 1.0, 0.0)
        msg_ref[...] = jnp.concatenate([msg, pad], axis=1)

    return pl.pallas_call(
        body,
        grid=(NE // EBLK,),
        in_specs=[
            pl.BlockSpec((EBLK, H), lambda i: (i, 0)),
            pl.BlockSpec((EBLK, W128), lambda i: (i, 0)),
            pl.BlockSpec((H, H * H), lambda i: (0, 0)),
            pl.BlockSpec((H, H), lambda i: (0, 0)),
        ],
        out_specs=pl.BlockSpec((EBLK, W128), lambda i: (i, 0)),
        out_shape=jax.ShapeDtypeStruct((NE, W128), jnp.float32),
    )(f, o_src, M, Be2)


def _tc_gru(s, agg2, W_root, b_conv, Wih, bih, Whh, bhh):
    R = s.shape[0]

    def body(s_ref, agg_ref, wr, bc, wih, bih_, whh, bhh_, out_ref):
        ag = agg_ref[...]
        both = ag[0] + ag[1]
        agg = both[:, :H]
        deg = both[:, H:H + 1]
        inv = 1.0 / jnp.maximum(deg, 1.0)
        s_ = s_ref[...][:, :H]
        m = _relu(_dot(s_, wr[...]) + agg * inv + bc[...])
        gx = _dot(m, wih[...]) + bih_[...]
        gh = _dot(s_, whh[...]) + bhh_[...]
        r = jax.nn.sigmoid(gx[:, :H] + gh[:, :H])
        zg = jax.nn.sigmoid(gx[:, H:2 * H] + gh[:, H:2 * H])
        n = jnp.tanh(gx[:, 2 * H:] + r * gh[:, 2 * H:])
        s_new = (1.0 - zg) * n + zg * s_
        out_ref[...] = jnp.concatenate(
            [s_new, jnp.zeros((R, W128 - H), jnp.float32)], axis=1)

    return pl.pallas_call(
        body,
        out_shape=jax.ShapeDtypeStruct((R, W128), jnp.float32),
    )(s, agg2, W_root, b_conv, Wih, bih, Whh, bhh)


def _tc_final(out_nodes, n_feat, Wih_s2s, Whh_s2s, b_s2s, Wih_m, b_m,
              Wm1a, Wm1b, bm1, Wm2, bm2):
    def body(o_ref, nf_ref, wihs, whhs, bs, wim, bm, w1a, w1b, b1, w2, b2,
             logit_ref, hx_ref, cx_ref):
        o3 = o_ref[...][:, :H].reshape(NG, NPG, H)
        q_star = jnp.zeros((NG, 2 * H), jnp.float32)
        hs = jnp.zeros((NG, H), jnp.float32)
        cs = jnp.zeros((NG, H), jnp.float32)
        for _ in range(6):
            gates = _dot(q_star, wihs[...]) + _dot(hs, whhs[...]) + bs[...]
            ig = jax.nn.sigmoid(gates[:, :H])
            fg = jax.nn.sigmoid(gates[:, H:2 * H])
            gg = jnp.tanh(gates[:, 2 * H:3 * H])
            og = jax.nn.sigmoid(gates[:, 3 * H:])
            cs = fg * cs + ig * gg
            hs = og * jnp.tanh(cs)
            e = jnp.sum(o3 * hs[:, None, :], axis=2)
            ex = jnp.exp(e - jnp.max(e, axis=1, keepdims=True))
            a = ex / jnp.sum(ex, axis=1, keepdims=True)
            rg = jnp.sum(a[:, :, None] * o3, axis=1)
            q_star = jnp.concatenate([hs, rg], axis=1)
        gates = _dot(q_star, wim[...]) + bm[...]
        ig = jax.nn.sigmoid(gates[:, :H])
        gg = jnp.tanh(gates[:, 2 * H:3 * H])
        og = jax.nn.sigmoid(gates[:, 3 * H:])
        cx = ig * gg
        hx = og * jnp.tanh(cx)
        hx_ref[...] = hx
        cx_ref[...] = cx
        t = _dot(hx, w1a[...])
        t4 = jnp.broadcast_to(t[:, None, :], (NG, TP, H)).reshape(NG * TP, H)
        hid = _relu(t4 + _dot(nf_ref[...], w1b[...]) + b1[...])
        logit_ref[...] = _dot(hid, w2[...]) + b2[...]

    return pl.pallas_call(
        body,
        out_shape=(jax.ShapeDtypeStruct((NG * TP, ACT), jnp.float32),
                   jax.ShapeDtypeStruct((NG, H), jnp.float32),
                   jax.ShapeDtypeStruct((NG, H), jnp.float32)),
    )(out_nodes, n_feat, Wih_s2s, Whh_s2s, b_s2s, Wih_m, b_m,
      Wm1a, Wm1b, bm1, Wm2, bm2)


def kernel(x, edge_attr, W_lin0, b_lin0, We1, be1, We2, be2, W_root, b_conv,
           Wih_gru, Whh_gru, bih_gru, bhh_gru, Wih_s2s, Whh_s2s, bih_s2s,
           bhh_s2s, Wih_m, Whh_m, bih_m, bhh_m, Wm1, bm1, Wm2, bm2,
           edge_index, batch, nonring, nrbidx):
    f32 = jnp.float32
    src = edge_index[0]
    dst = edge_index[1]
    M = We2.reshape(H, H, H).transpose(1, 0, 2).reshape(H, H * H)
    Be2 = be2.reshape(H, H)
    zeros_agg = jnp.zeros((NN, W128), f32)

    gru_w = (W_root, b_conv.reshape(1, H), Wih_gru,
             bih_gru.reshape(1, 3 * H), Whh_gru, bhh_gru.reshape(1, 3 * H))
    s, f = _tc_prep(x, W_lin0, b_lin0.reshape(1, H), edge_attr, We1,
                    be1.reshape(1, H))
    for t in range(6):
        T = _sc_gather(s, src, W128)
        msg = _tc_msg(f, T, M, Be2)
        agg2 = _sc_scatter_add(msg, dst, zeros_agg)
        s = _tc_gru(s, agg2, *gru_w)
    nf = _sc_gather(s, nonring.reshape(-1), W128)[:, :H].reshape(NG * TP, 4 * H)
    logits, hx, cx = _tc_final(
        s, nf, Wih_s2s, Whh_s2s, (bih_s2s + bhh_s2s).reshape(1, 4 * H),
        Wih_m, (bih_m + bhh_m).reshape(1, 4 * H),
        Wm1[:H], Wm1[H:], bm1.reshape(1, H), Wm2, bm2.reshape(1, ACT))
    return (logits.reshape(NG, TP, ACT), hx[None], cx[None])
